# Initial kernel scaffold; baseline (speedup 1.0000x reference)
import jax
import jax.numpy as jnp
from jax.experimental import pallas as pl


def kernel(x, edge_index, Wr, Wi, fc_w, fc_b):
    raise NotImplementedError("write your pallas kernel here")


# speedup vs baseline: 192.9339x; 192.9339x over previous
import functools

import numpy as np
import jax
import jax.numpy as jnp
from jax import lax
from jax.experimental import pallas as pl
from jax.experimental.pallas import tpu as pltpu
from jax.experimental.pallas import tpu_sc as plsc

_N = 50000
_E = 1600000
_H = 15
_FQ = 8
_K = 6
_COUT = 10
_C = 6
_P = _H * _COUT

_NC = 2
_NS = 16
_NW = _NC * _NS
_EPW = _E // _NW
_CH = 2000
_NCH = _EPW // _CH
_ZCH = _N // _CH


def _sc_degrees(src_hbm, dst_hbm, out_hbm, idx_v, ones_v, zero_v,
                dego_sh, degi_sh):
    cid = lax.axis_index("c")
    sid = lax.axis_index("s")
    wid = cid * _NS + sid

    def _fill(i, _):
        ones_v[pl.ds(i * 16, 16)] = jnp.ones((16,), jnp.float32)
        zero_v[pl.ds(i * 16, 16)] = jnp.zeros((16,), jnp.float32)
        return 0
    lax.fori_loop(0, _CH // 16, _fill, 0)

    def _zero(cdx, _):
        @pl.when((cdx % _NS) == sid)
        def _():
            pltpu.sync_copy(zero_v, dego_sh.at[pl.ds(cdx * _CH, _CH)])
            pltpu.sync_copy(zero_v, degi_sh.at[pl.ds(cdx * _CH, _CH)])
        return 0
    lax.fori_loop(0, _ZCH, _zero, 0)
    plsc.subcore_barrier()

    base = wid * _EPW

    def _body(t, _):
        off = base + t * _CH
        pltpu.sync_copy(src_hbm.at[pl.ds(off, _CH)], idx_v)
        pltpu.sync_copy(ones_v, dego_sh.at[idx_v], add=True)
        pltpu.sync_copy(dst_hbm.at[pl.ds(off, _CH)], idx_v)
        pltpu.sync_copy(ones_v, degi_sh.at[idx_v], add=True)
        return 0
    lax.fori_loop(0, _NCH, _body, 0)
    plsc.subcore_barrier()

    def _read(cdx, _):
        @pl.when((cdx % _NS) == sid)
        def _():
            sl = pl.ds(cdx * _CH, _CH)
            obase = cid * (2 * _N) + cdx * _CH
            pltpu.sync_copy(dego_sh.at[sl], zero_v)
            pltpu.sync_copy(zero_v, out_hbm.at[pl.ds(obase, _CH)])
            pltpu.sync_copy(degi_sh.at[sl], zero_v)
            pltpu.sync_copy(zero_v, out_hbm.at[pl.ds(obase + _N, _CH)])
        return 0
    lax.fori_loop(0, _ZCH, _read, 0)


def _sc_spmv(u_hbm, src_hbm, dst_hbm, out_hbm, src_v, dst_v, rows_v,
             zrows_v, acc_sh, sem):
    cid = lax.axis_index("c")
    sid = lax.axis_index("s")
    wid = cid * _NS + sid

    def _zfill(i, _):
        zrows_v[i] = jnp.zeros((16,), jnp.float32)
        return 0
    lax.fori_loop(0, _CH, _zfill, 0)

    def _zero(cdx, _):
        @pl.when((cdx % _NS) == sid)
        def _():
            pltpu.sync_copy(zrows_v, acc_sh.at[pl.ds(cdx * _CH, _CH)])
        return 0
    lax.fori_loop(0, _ZCH, _zero, 0)
    plsc.subcore_barrier()

    base = wid * _EPW

    def _body(t, _):
        off = base + t * _CH
        pltpu.sync_copy(src_hbm.at[pl.ds(off, _CH)], src_v)
        pltpu.sync_copy(dst_hbm.at[pl.ds(off, _CH)], dst_v)
        pltpu.async_copy(u_hbm.at[src_v], rows_v, sem).wait()
        pltpu.sync_copy(rows_v, acc_sh.at[dst_v], add=True)
        return 0
    lax.fori_loop(0, _NCH, _body, 0)
    plsc.subcore_barrier()

    def _read(cdx, _):
        @pl.when((cdx % _NS) == sid)
        def _():
            sl = pl.ds(cdx * _CH, _CH)
            pltpu.sync_copy(acc_sh.at[sl], zrows_v)
            pltpu.sync_copy(zrows_v, out_hbm.at[cid, sl])
        return 0
    lax.fori_loop(0, _ZCH, _read, 0)


_sc_mesh = plsc.VectorSubcoreMesh(core_axis_name="c", subcore_axis_name="s")
_sc_params = pltpu.CompilerParams(use_tc_tiling_on_sc=False)

_deg_call = pl.kernel(
    _sc_degrees,
    out_type=jax.ShapeDtypeStruct((_NC * 2 * _N,), jnp.float32),
    mesh=_sc_mesh,
    compiler_params=_sc_params,
    scratch_types=[
        pltpu.VMEM((_CH,), jnp.int32),
        pltpu.VMEM((_CH,), jnp.float32),
        pltpu.VMEM((_CH,), jnp.float32),
        pltpu.VMEM_SHARED((_N,), jnp.float32),
        pltpu.VMEM_SHARED((_N,), jnp.float32),
    ],
)

_spmv_call = pl.kernel(
    _sc_spmv,
    out_type=jax.ShapeDtypeStruct((_NC, _N, 16), jnp.float32),
    mesh=_sc_mesh,
    compiler_params=_sc_params,
    scratch_types=[
        pltpu.VMEM((_CH,), jnp.int32),
        pltpu.VMEM((_CH,), jnp.int32),
        pltpu.VMEM((_CH, 16), jnp.float32),
        pltpu.VMEM((_CH, 16), jnp.float32),
        pltpu.VMEM_SHARED((_N, 16), jnp.float32),
        pltpu.SemaphoreType.DMA,
    ],
)


def _tc_prep(degp_ref, x_ref, dmat_ref, rs_ref, t0_ref, u0_ref):
    deg_o = jnp.maximum(degp_ref[0, 0] + degp_ref[1, 0], 1.0)
    deg_i = jnp.maximum(degp_ref[0, 1] + degp_ref[1, 1], 1.0)
    rs_o = lax.rsqrt(deg_o)
    rs_i = lax.rsqrt(deg_i)
    rs_ref[0] = rs_o
    rs_ref[1] = rs_i
    t0 = jnp.dot(x_ref[...], dmat_ref[...], preferred_element_type=jnp.float32)
    t0_ref[...] = t0
    u0_ref[...] = rs_o * t0


def _tc_step(p_ref, rs_ref, tm2_ref, t_ref, u_ref, *, c2, cm):
    s = -rs_ref[1] * (p_ref[0] + p_ref[1])
    t = c2 * s - cm * tm2_ref[...]
    t_ref[...] = t
    u_ref[...] = rs_ref[0] * t


_BN = 2000
_NB = _N // _BN


def _tc_final(b_ref, fcb_ref, t0_ref, t1_ref, t2_ref, t3_ref, t4_ref,
              t5_ref, fc3_ref, out_ref):
    i = pl.program_id(0)

    @pl.when(i == 0)
    def _():
        out_ref[...] = jnp.zeros_like(out_ref)

    trefs = (t0_ref, t1_ref, t2_ref, t3_ref, t4_ref, t5_ref)
    y = jnp.dot(trefs[0][...], b_ref[0],
                preferred_element_type=jnp.float32)
    for k in range(1, _K):
        y = y + jnp.dot(trefs[k][...], b_ref[k],
                        preferred_element_type=jnp.float32)
    y = jnp.maximum(y, 0.0)
    contrib = jnp.sum(fc3_ref[...] * y[None, :, :], axis=(1, 2))
    out_ref[...] += contrib

    @pl.when(i == _NB - 1)
    def _():
        logits = out_ref[...] + fcb_ref[...]
        m = jnp.max(logits)
        ls = logits - m
        out_ref[...] = ls - jnp.log(jnp.sum(jnp.exp(ls)))


_SBN = 5000
_SNB = _N // _SBN

_prep_call = pl.pallas_call(
    _tc_prep,
    grid=(_SNB,),
    in_specs=[
        pl.BlockSpec((2, 2, _SBN, 1), lambda i: (0, 0, i, 0)),
        pl.BlockSpec((_SBN, _H), lambda i: (i, 0)),
        pl.BlockSpec((_H, 16), lambda i: (0, 0)),
    ],
    out_specs=[
        pl.BlockSpec((2, _SBN, 1), lambda i: (0, i, 0)),
        pl.BlockSpec((_SBN, 16), lambda i: (i, 0)),
        pl.BlockSpec((_SBN, 16), lambda i: (i, 0)),
    ],
    out_shape=[
        jax.ShapeDtypeStruct((2, _N, 1), jnp.float32),
        jax.ShapeDtypeStruct((_N, 16), jnp.float32),
        jax.ShapeDtypeStruct((_N, 16), jnp.float32),
    ],
)


def _make_step(c2, cm):
    return pl.pallas_call(
        functools.partial(_tc_step, c2=c2, cm=cm),
        grid=(_SNB,),
        in_specs=[
            pl.BlockSpec((2, _SBN, 16), lambda i: (0, i, 0)),
            pl.BlockSpec((2, _SBN, 1), lambda i: (0, i, 0)),
            pl.BlockSpec((_SBN, 16), lambda i: (i, 0)),
        ],
        out_specs=[
            pl.BlockSpec((_SBN, 16), lambda i: (i, 0)),
            pl.BlockSpec((_SBN, 16), lambda i: (i, 0)),
        ],
        out_shape=[
            jax.ShapeDtypeStruct((_N, 16), jnp.float32),
            jax.ShapeDtypeStruct((_N, 16), jnp.float32),
        ],
    )


_step1_call = _make_step(1.0, 0.0)
_stepk_call = _make_step(2.0, 1.0)

_final_call = pl.pallas_call(
    _tc_final,
    grid=(_NB,),
    in_specs=[
        pl.BlockSpec((_K, 16, _P), lambda i: (0, 0, 0)),
        pl.BlockSpec((_C,), lambda i: (0,)),
    ] + [pl.BlockSpec((_BN, 16), lambda i: (i, 0)) for _ in range(_K)] + [
        pl.BlockSpec((_C, _BN, _P), lambda i: (0, i, 0)),
    ],
    out_specs=pl.BlockSpec((_C,), lambda i: (0,)),
    out_shape=jax.ShapeDtypeStruct((_C,), jnp.float32),
)


def _build_mats(Wr, Wi):
    h = np.arange(_H, dtype=np.float64)[:, None]
    f = np.arange(_FQ, dtype=np.float64)[None, :]
    ang = 2.0 * np.pi * h * f / _H
    dmat = jnp.asarray(
        np.concatenate([np.cos(ang), -np.sin(ang)], axis=1), jnp.float32)

    ang2 = ang.T
    w2 = np.where(np.arange(_FQ)[:, None] == 0, 1.0, 2.0)
    cr = jnp.asarray((w2 * np.cos(ang2)) / _H, jnp.float32)
    ci_np = -(w2 * np.sin(ang2)) / _H
    ci_np[0, :] = 0.0
    ci = jnp.asarray(ci_np, jnp.float32)

    wr = Wr[:, :, 0, :]
    wi = Wi[:, :, 0, :]
    b_re = (jnp.einsum('fh,kfo->kfho', cr, wr)
            + jnp.einsum('fh,kfo->kfho', ci, wi))
    b_im = (-jnp.einsum('fh,kfo->kfho', cr, wi)
            + jnp.einsum('fh,kfo->kfho', ci, wr))
    bmat = jnp.concatenate([b_re, b_im], axis=1)
    return dmat, bmat.reshape(_K, 16, _P)


def kernel(x, edge_index, Wr, Wi, fc_w, fc_b):
    x2 = x[:, :, 0]
    src = edge_index[0]
    dst = edge_index[1]
    dmat, bmat = _build_mats(Wr, Wi)

    degp = _deg_call(src, dst).reshape(_NC, 2, _N, 1)
    rs, t0, u = _prep_call(degp, x2, dmat)

    ts = [t0]
    for k in range(1, _K):
        p = _spmv_call(u, src, dst)
        call = _step1_call if k == 1 else _stepk_call
        t, u = call(p, rs, ts[-2] if k >= 2 else t0)
        ts.append(t)

    fc3 = fc_w.reshape(_C, _N, _P)
    return _final_call(bmat, fc_b, *ts, fc3)


# speedup vs baseline: 575.0607x; 2.9806x over previous
import functools

import numpy as np
import jax
import jax.numpy as jnp
from jax import lax
from jax.experimental import pallas as pl
from jax.experimental.pallas import tpu as pltpu
from jax.experimental.pallas import tpu_sc as plsc

_N = 50000
_E = 1600000
_H = 15
_FQ = 8
_K = 6
_COUT = 10
_C = 6
_P = _H * _COUT

_NC = 2
_NS = 16
_NW = _NC * _NS
_EPW = _E // _NW
_CH = 2000
_NCH = _EPW // _CH
_ZCH = _N // _CH


def _sc_degrees(src_hbm, dst_hbm, out_hbm, idx_v, ones_v, zero_v,
                dego_sh, degi_sh):
    cid = lax.axis_index("c")
    sid = lax.axis_index("s")
    wid = cid * _NS + sid

    def _fill(i, _):
        ones_v[pl.ds(i * 16, 16)] = jnp.ones((16,), jnp.float32)
        zero_v[pl.ds(i * 16, 16)] = jnp.zeros((16,), jnp.float32)
        return 0
    lax.fori_loop(0, _CH // 16, _fill, 0)

    def _zero(cdx, _):
        @pl.when((cdx % _NS) == sid)
        def _():
            pltpu.sync_copy(zero_v, dego_sh.at[pl.ds(cdx * _CH, _CH)])
            pltpu.sync_copy(zero_v, degi_sh.at[pl.ds(cdx * _CH, _CH)])
        return 0
    lax.fori_loop(0, _ZCH, _zero, 0)
    plsc.subcore_barrier()

    base = wid * _EPW

    def _body(t, _):
        off = base + t * _CH
        pltpu.sync_copy(src_hbm.at[pl.ds(off, _CH)], idx_v)
        pltpu.sync_copy(ones_v, dego_sh.at[idx_v], add=True)
        pltpu.sync_copy(dst_hbm.at[pl.ds(off, _CH)], idx_v)
        pltpu.sync_copy(ones_v, degi_sh.at[idx_v], add=True)
        return 0
    lax.fori_loop(0, _NCH, _body, 0)
    plsc.subcore_barrier()

    def _read(cdx, _):
        @pl.when((cdx % _NS) == sid)
        def _():
            sl = pl.ds(cdx * _CH, _CH)
            obase = cid * (2 * _N) + cdx * _CH
            pltpu.sync_copy(dego_sh.at[sl], zero_v)
            pltpu.sync_copy(zero_v, out_hbm.at[pl.ds(obase, _CH)])
            pltpu.sync_copy(degi_sh.at[sl], zero_v)
            pltpu.sync_copy(zero_v, out_hbm.at[pl.ds(obase + _N, _CH)])
        return 0
    lax.fori_loop(0, _ZCH, _read, 0)


def _sc_spmv(u_hbm, src_hbm, dst_hbm, out_hbm, src_v, dst_v, rows_v,
             zrows_v, acc_sh, sem):
    cid = lax.axis_index("c")
    sid = lax.axis_index("s")
    wid = cid * _NS + sid

    def _zfill(i, _):
        zrows_v[i] = jnp.zeros((16,), jnp.float32)
        return 0
    lax.fori_loop(0, _CH, _zfill, 0)

    def _zero(cdx, _):
        @pl.when((cdx % _NS) == sid)
        def _():
            pltpu.sync_copy(zrows_v, acc_sh.at[pl.ds(cdx * _CH, _CH)])
        return 0
    lax.fori_loop(0, _ZCH, _zero, 0)
    plsc.subcore_barrier()

    base = wid * _EPW

    def _body(t, _):
        off = base + t * _CH
        pltpu.sync_copy(src_hbm.at[pl.ds(off, _CH)], src_v)
        pltpu.sync_copy(dst_hbm.at[pl.ds(off, _CH)], dst_v)
        pltpu.async_copy(u_hbm.at[src_v], rows_v, sem).wait()
        pltpu.sync_copy(rows_v, acc_sh.at[dst_v], add=True)
        return 0
    lax.fori_loop(0, _NCH, _body, 0)
    plsc.subcore_barrier()

    def _read(cdx, _):
        @pl.when((cdx % _NS) == sid)
        def _():
            sl = pl.ds(cdx * _CH, _CH)
            pltpu.sync_copy(acc_sh.at[sl], zrows_v)
            pltpu.sync_copy(zrows_v, out_hbm.at[cid, sl])
        return 0
    lax.fori_loop(0, _ZCH, _read, 0)


_sc_mesh = plsc.VectorSubcoreMesh(core_axis_name="c", subcore_axis_name="s")
_sc_params = pltpu.CompilerParams(use_tc_tiling_on_sc=False)

_deg_call = pl.kernel(
    _sc_degrees,
    out_type=jax.ShapeDtypeStruct((_NC * 2 * _N,), jnp.float32),
    mesh=_sc_mesh,
    compiler_params=_sc_params,
    scratch_types=[
        pltpu.VMEM((_CH,), jnp.int32),
        pltpu.VMEM((_CH,), jnp.float32),
        pltpu.VMEM((_CH,), jnp.float32),
        pltpu.VMEM_SHARED((_N,), jnp.float32),
        pltpu.VMEM_SHARED((_N,), jnp.float32),
    ],
)

_spmv_call = pl.kernel(
    _sc_spmv,
    out_type=jax.ShapeDtypeStruct((_NC, _N, 16), jnp.float32),
    mesh=_sc_mesh,
    compiler_params=_sc_params,
    scratch_types=[
        pltpu.VMEM((_CH,), jnp.int32),
        pltpu.VMEM((_CH,), jnp.int32),
        pltpu.VMEM((_CH, 16), jnp.float32),
        pltpu.VMEM((_CH, 16), jnp.float32),
        pltpu.VMEM_SHARED((_N, 16), jnp.float32),
        pltpu.SemaphoreType.DMA,
    ],
)


def _tc_prep(degp_ref, x_ref, dmat_ref, rs_ref, t0_ref, u0_ref):
    deg_o = jnp.maximum(degp_ref[0, 0] + degp_ref[1, 0], 1.0)
    deg_i = jnp.maximum(degp_ref[0, 1] + degp_ref[1, 1], 1.0)
    rs_o = lax.rsqrt(deg_o)
    rs_i = lax.rsqrt(deg_i)
    rs_ref[0] = rs_o
    rs_ref[1] = rs_i
    t0 = jnp.dot(x_ref[...], dmat_ref[...], preferred_element_type=jnp.float32)
    t0_ref[...] = t0
    u0_ref[...] = rs_o * t0


def _tc_step(p_ref, rs_ref, tm2_ref, t_ref, u_ref, *, c2, cm):
    s = -rs_ref[1] * (p_ref[0] + p_ref[1])
    t = c2 * s - cm * tm2_ref[...]
    t_ref[...] = t
    u_ref[...] = rs_ref[0] * t


_BN = 2000
_NB = _N // _BN


def _tc_y(b_ref, t0_ref, t1_ref, t2_ref, t3_ref, t4_ref, t5_ref, y_ref):
    trefs = (t0_ref, t1_ref, t2_ref, t3_ref, t4_ref, t5_ref)
    y = jnp.dot(trefs[0][...], b_ref[0],
                preferred_element_type=jnp.float32)
    for k in range(1, _K):
        y = y + jnp.dot(trefs[k][...], b_ref[k],
                        preferred_element_type=jnp.float32)
    y_ref[...] = jnp.maximum(y, 0.0)


_FCB = 245760
_FNB = (_N * _P + _FCB - 1) // _FCB
_FC_TAIL = _N * _P - (_FNB - 1) * _FCB


def _tc_fc(fcb_ref, y_ref, fc_ref, out_ref):
    i = pl.program_id(0)

    @pl.when(i == 0)
    def _():
        out_ref[...] = jnp.zeros_like(out_ref)

    valid = jnp.where(i == _FNB - 1, _FC_TAIL, _FCB)
    mask = lax.broadcasted_iota(jnp.int32, (1, _FCB), 1) < valid
    prod = fc_ref[...] * y_ref[...][None, :]
    prod = jnp.where(mask, prod, 0.0)
    out_ref[...] += jnp.sum(prod, axis=1)

    @pl.when(i == _FNB - 1)
    def _():
        logits = out_ref[...] + fcb_ref[...]
        m = jnp.max(logits)
        ls = logits - m
        out_ref[...] = ls - jnp.log(jnp.sum(jnp.exp(ls)))


_SBN = 5000
_SNB = _N // _SBN

_prep_call = pl.pallas_call(
    _tc_prep,
    grid=(_SNB,),
    in_specs=[
        pl.BlockSpec((2, 2, _SBN, 1), lambda i: (0, 0, i, 0)),
        pl.BlockSpec((_SBN, _H), lambda i: (i, 0)),
        pl.BlockSpec((_H, 16), lambda i: (0, 0)),
    ],
    out_specs=[
        pl.BlockSpec((2, _SBN, 1), lambda i: (0, i, 0)),
        pl.BlockSpec((_SBN, 16), lambda i: (i, 0)),
        pl.BlockSpec((_SBN, 16), lambda i: (i, 0)),
    ],
    out_shape=[
        jax.ShapeDtypeStruct((2, _N, 1), jnp.float32),
        jax.ShapeDtypeStruct((_N, 16), jnp.float32),
        jax.ShapeDtypeStruct((_N, 16), jnp.float32),
    ],
)


def _make_step(c2, cm):
    return pl.pallas_call(
        functools.partial(_tc_step, c2=c2, cm=cm),
        grid=(_SNB,),
        in_specs=[
            pl.BlockSpec((2, _SBN, 16), lambda i: (0, i, 0)),
            pl.BlockSpec((2, _SBN, 1), lambda i: (0, i, 0)),
            pl.BlockSpec((_SBN, 16), lambda i: (i, 0)),
        ],
        out_specs=[
            pl.BlockSpec((_SBN, 16), lambda i: (i, 0)),
            pl.BlockSpec((_SBN, 16), lambda i: (i, 0)),
        ],
        out_shape=[
            jax.ShapeDtypeStruct((_N, 16), jnp.float32),
            jax.ShapeDtypeStruct((_N, 16), jnp.float32),
        ],
    )


_step1_call = _make_step(1.0, 0.0)
_stepk_call = _make_step(2.0, 1.0)

_y_call = pl.pallas_call(
    _tc_y,
    grid=(_NB,),
    in_specs=[
        pl.BlockSpec((_K, 16, _P), lambda i: (0, 0, 0)),
    ] + [pl.BlockSpec((_BN, 16), lambda i: (i, 0)) for _ in range(_K)],
    out_specs=pl.BlockSpec((_BN, _P), lambda i: (i, 0)),
    out_shape=jax.ShapeDtypeStruct((_N, _P), jnp.float32),
)

_fc_call = pl.pallas_call(
    _tc_fc,
    grid=(_FNB,),
    in_specs=[
        pl.BlockSpec((_C,), lambda i: (0,)),
        pl.BlockSpec((_FCB,), lambda i: (i,)),
        pl.BlockSpec((_C, _FCB), lambda i: (0, i)),
    ],
    out_specs=pl.BlockSpec((_C,), lambda i: (0,)),
    out_shape=jax.ShapeDtypeStruct((_C,), jnp.float32),
)


def _build_mats(Wr, Wi):
    h = np.arange(_H, dtype=np.float64)[:, None]
    f = np.arange(_FQ, dtype=np.float64)[None, :]
    ang = 2.0 * np.pi * h * f / _H
    dmat = jnp.asarray(
        np.concatenate([np.cos(ang), -np.sin(ang)], axis=1), jnp.float32)

    ang2 = ang.T
    w2 = np.where(np.arange(_FQ)[:, None] == 0, 1.0, 2.0)
    cr = jnp.asarray((w2 * np.cos(ang2)) / _H, jnp.float32)
    ci_np = -(w2 * np.sin(ang2)) / _H
    ci_np[0, :] = 0.0
    ci = jnp.asarray(ci_np, jnp.float32)

    wr = Wr[:, :, 0, :]
    wi = Wi[:, :, 0, :]
    b_re = (jnp.einsum('fh,kfo->kfho', cr, wr)
            + jnp.einsum('fh,kfo->kfho', ci, wi))
    b_im = (-jnp.einsum('fh,kfo->kfho', cr, wi)
            + jnp.einsum('fh,kfo->kfho', ci, wr))
    bmat = jnp.concatenate([b_re, b_im], axis=1)
    return dmat, bmat.reshape(_K, 16, _P)


def kernel(x, edge_index, Wr, Wi, fc_w, fc_b):
    x2 = x[:, :, 0]
    src = edge_index[0]
    dst = edge_index[1]
    dmat, bmat = _build_mats(Wr, Wi)

    degp = _deg_call(src, dst).reshape(_NC, 2, _N, 1)
    rs, t0, u = _prep_call(degp, x2, dmat)

    ts = [t0]
    for k in range(1, _K):
        p = _spmv_call(u, src, dst)
        call = _step1_call if k == 1 else _stepk_call
        t, u = call(p, rs, ts[-2] if k >= 2 else t0)
        ts.append(t)

    y = _y_call(bmat, *ts)
    return _fc_call(fc_b, y.reshape(-1), fc_w)


# speedup vs baseline: 909.8860x; 1.5822x over previous
import functools

import numpy as np
import jax
import jax.numpy as jnp
from jax import lax
from jax.experimental import pallas as pl
from jax.experimental.pallas import tpu as pltpu
from jax.experimental.pallas import tpu_sc as plsc

_N = 50000
_E = 1600000
_H = 15
_FQ = 8
_K = 6
_COUT = 10
_C = 6
_P = _H * _COUT

_NC = 2
_NS = 16
_NW = _NC * _NS
_EPW = _E // _NW
_CH = 2000
_NCH = _EPW // _CH
_ZCH = _N // _CH


def _sc_degrees(src_hbm, dst_hbm, out_hbm, idx_v, ones_v, zero_v,
                dego_sh, degi_sh):
    cid = lax.axis_index("c")
    sid = lax.axis_index("s")
    wid = cid * _NS + sid

    def _fill(i, _):
        ones_v[pl.ds(i * 16, 16)] = jnp.ones((16,), jnp.float32)
        zero_v[pl.ds(i * 16, 16)] = jnp.zeros((16,), jnp.float32)
        return 0
    lax.fori_loop(0, _CH // 16, _fill, 0)

    def _zero(cdx, _):
        @pl.when((cdx % _NS) == sid)
        def _():
            pltpu.sync_copy(zero_v, dego_sh.at[pl.ds(cdx * _CH, _CH)])
            pltpu.sync_copy(zero_v, degi_sh.at[pl.ds(cdx * _CH, _CH)])
        return 0
    lax.fori_loop(0, _ZCH, _zero, 0)
    plsc.subcore_barrier()

    base = wid * _EPW

    def _body(t, _):
        off = base + t * _CH
        pltpu.sync_copy(src_hbm.at[pl.ds(off, _CH)], idx_v)
        pltpu.sync_copy(ones_v, dego_sh.at[idx_v], add=True)
        pltpu.sync_copy(dst_hbm.at[pl.ds(off, _CH)], idx_v)
        pltpu.sync_copy(ones_v, degi_sh.at[idx_v], add=True)
        return 0
    lax.fori_loop(0, _NCH, _body, 0)
    plsc.subcore_barrier()

    def _read(cdx, _):
        @pl.when((cdx % _NS) == sid)
        def _():
            sl = pl.ds(cdx * _CH, _CH)
            obase = cid * (2 * _N) + cdx * _CH
            pltpu.sync_copy(dego_sh.at[sl], zero_v)
            pltpu.sync_copy(zero_v, out_hbm.at[pl.ds(obase, _CH)])
            pltpu.sync_copy(degi_sh.at[sl], zero_v)
            pltpu.sync_copy(zero_v, out_hbm.at[pl.ds(obase + _N, _CH)])
        return 0
    lax.fori_loop(0, _ZCH, _read, 0)


def _sc_spmv(u_hbm, src_hbm, dst_hbm, out_hbm,
             src0, src1, src2, dst0, dst1, dst2, rows0, rows1,
             acc_sh, is0, is1, is2, gs0, gs1, ss0, ss1):
    cid = lax.axis_index("c")
    sid = lax.axis_index("s")
    wid = cid * _NS + sid
    srcs = (src0, src1, src2)
    dsts = (dst0, dst1, dst2)
    rows = (rows0, rows1)
    isems = (is0, is1, is2)
    gsems = (gs0, gs1)
    ssems = (ss0, ss1)

    def _zfill(i, _):
        rows0[i] = jnp.zeros((16,), jnp.float32)
        return 0
    lax.fori_loop(0, _CH, _zfill, 0)

    def _zero(cdx, _):
        @pl.when((cdx % _NS) == sid)
        def _():
            pltpu.sync_copy(rows0, acc_sh.at[pl.ds(cdx * _CH, _CH)])
        return 0
    lax.fori_loop(0, _ZCH, _zero, 0)
    plsc.subcore_barrier()

    base = wid * _EPW

    def _idx_start(t, j):
        off = base + t * _CH
        pltpu.async_copy(src_hbm.at[pl.ds(off, _CH)], srcs[j], isems[j])
        pltpu.async_copy(dst_hbm.at[pl.ds(off, _CH)], dsts[j], isems[j])

    def _idx_wait(j):
        pltpu.make_async_copy(src_hbm.at[pl.ds(0, _CH)], srcs[j],
                              isems[j]).wait()
        pltpu.make_async_copy(dst_hbm.at[pl.ds(0, _CH)], dsts[j],
                              isems[j]).wait()

    def _gather_start(j, b):
        pltpu.async_copy(u_hbm.at[srcs[j]], rows[b], gsems[b])

    def _gather_wait(b):
        pltpu.make_async_copy(u_hbm.at[pl.ds(0, _CH)], rows[b],
                              gsems[b]).wait()

    def _scatter_start(j, b):
        pltpu.async_copy(rows[b], acc_sh.at[dsts[j]], ssems[b], add=True)

    def _scatter_wait(b):
        pltpu.make_async_copy(u_hbm.at[pl.ds(0, _CH)], rows[b],
                              ssems[b]).wait()

    _idx_start(0, 0)
    _idx_start(1, 1)
    _idx_wait(0)
    _gather_start(0, 0)
    _idx_start(2, 2)

    _gather_wait(0)
    _scatter_start(0, 0)
    _idx_wait(1)
    _gather_start(1, 1)

    for t in range(1, _NCH - 1):
        b, nb = t % 2, (t + 1) % 2
        jn, j2 = (t + 1) % 3, (t + 2) % 3
        _gather_wait(b)
        _scatter_start(t % 3, b)
        _scatter_wait(nb)
        _idx_wait(jn)
        _gather_start(jn, nb)
        if t + 2 < _NCH:
            _idx_start(t + 2, j2)

    _lb = (_NCH - 1) % 2
    _gather_wait(_lb)
    _scatter_start((_NCH - 1) % 3, _lb)
    _scatter_wait((_NCH - 1 + 1) % 2)
    _scatter_wait(_lb)
    plsc.subcore_barrier()

    def _read(cdx, _):
        @pl.when((cdx % _NS) == sid)
        def _():
            sl = pl.ds(cdx * _CH, _CH)
            pltpu.sync_copy(acc_sh.at[sl], rows0)
            pltpu.sync_copy(rows0, out_hbm.at[cid, sl])
        return 0
    lax.fori_loop(0, _ZCH, _read, 0)


_sc_mesh = plsc.VectorSubcoreMesh(core_axis_name="c", subcore_axis_name="s")
_sc_params = pltpu.CompilerParams(use_tc_tiling_on_sc=False)

_deg_call = pl.kernel(
    _sc_degrees,
    out_type=jax.ShapeDtypeStruct((_NC * 2 * _N,), jnp.float32),
    mesh=_sc_mesh,
    compiler_params=_sc_params,
    scratch_types=[
        pltpu.VMEM((_CH,), jnp.int32),
        pltpu.VMEM((_CH,), jnp.float32),
        pltpu.VMEM((_CH,), jnp.float32),
        pltpu.VMEM_SHARED((_N,), jnp.float32),
        pltpu.VMEM_SHARED((_N,), jnp.float32),
    ],
)

_spmv_call = pl.kernel(
    _sc_spmv,
    out_type=jax.ShapeDtypeStruct((_NC, _N, 16), jnp.float32),
    mesh=_sc_mesh,
    compiler_params=_sc_params,
    scratch_types=(
        [pltpu.VMEM((_CH,), jnp.int32) for _ in range(6)]
        + [pltpu.VMEM((_CH, 16), jnp.float32) for _ in range(2)]
        + [pltpu.VMEM_SHARED((_N, 16), jnp.float32)]
        + [pltpu.SemaphoreType.DMA for _ in range(7)]
    ),
)


def _tc_prep(degp_ref, x_ref, dmat_ref, rso_ref, rsi_ref, t0_ref, u0_ref):
    deg_o = jnp.maximum(degp_ref[0, 0] + degp_ref[1, 0], 1.0)
    deg_i = jnp.maximum(degp_ref[0, 1] + degp_ref[1, 1], 1.0)
    rs_o = lax.rsqrt(deg_o)
    rs_i = lax.rsqrt(deg_i)
    rso_ref[...] = jnp.broadcast_to(rs_o, rso_ref.shape)
    rsi_ref[...] = jnp.broadcast_to(rs_i, rsi_ref.shape)
    t0 = jnp.dot(x_ref[...], dmat_ref[...], preferred_element_type=jnp.float32)
    t0_ref[...] = t0
    u0_ref[...] = rs_o * t0


def _tc_step(p_ref, rsi_ref, rso_ref, tm2_ref, t_ref, u_ref, *, c2, cm):
    s = -rsi_ref[...] * (p_ref[0] + p_ref[1])
    t = c2 * s - cm * tm2_ref[...]
    t_ref[...] = t
    u_ref[...] = rso_ref[...] * t


_BN = 2000
_NB = _N // _BN


def _tc_y(b_ref, t0_ref, t1_ref, t2_ref, t3_ref, t4_ref, t5_ref, y_ref):
    trefs = (t0_ref, t1_ref, t2_ref, t3_ref, t4_ref, t5_ref)
    y = jnp.dot(trefs[0][...], b_ref[0],
                preferred_element_type=jnp.float32)
    for k in range(1, _K):
        y = y + jnp.dot(trefs[k][...], b_ref[k],
                        preferred_element_type=jnp.float32)
    y_ref[...] = jnp.maximum(y, 0.0)


_FCB = 245760
_FNB = (_N * _P + _FCB - 1) // _FCB
_FC_TAIL = _N * _P - (_FNB - 1) * _FCB


def _tc_fc(fcb_ref, y_ref, fc_ref, out_ref):
    i = pl.program_id(0)

    @pl.when(i == 0)
    def _():
        out_ref[...] = jnp.zeros_like(out_ref)

    valid = jnp.where(i == _FNB - 1, _FC_TAIL, _FCB)
    mask = lax.broadcasted_iota(jnp.int32, (1, _FCB), 1) < valid
    prod = fc_ref[...] * y_ref[...][None, :]
    prod = jnp.where(mask, prod, 0.0)
    out_ref[...] += jnp.sum(prod, axis=1)

    @pl.when(i == _FNB - 1)
    def _():
        logits = out_ref[...] + fcb_ref[...]
        m = jnp.max(logits)
        ls = logits - m
        out_ref[...] = ls - jnp.log(jnp.sum(jnp.exp(ls)))


_SBN = 5000
_SNB = _N // _SBN

_prep_call = pl.pallas_call(
    _tc_prep,
    grid=(_SNB,),
    in_specs=[
        pl.BlockSpec((2, 2, _SBN, 1), lambda i: (0, 0, i, 0)),
        pl.BlockSpec((_SBN, _H), lambda i: (i, 0)),
        pl.BlockSpec((_H, 16), lambda i: (0, 0)),
    ],
    out_specs=[
        pl.BlockSpec((_SBN, 16), lambda i: (i, 0)),
        pl.BlockSpec((_SBN, 16), lambda i: (i, 0)),
        pl.BlockSpec((_SBN, 16), lambda i: (i, 0)),
        pl.BlockSpec((_SBN, 16), lambda i: (i, 0)),
    ],
    out_shape=[
        jax.ShapeDtypeStruct((_N, 16), jnp.float32),
        jax.ShapeDtypeStruct((_N, 16), jnp.float32),
        jax.ShapeDtypeStruct((_N, 16), jnp.float32),
        jax.ShapeDtypeStruct((_N, 16), jnp.float32),
    ],
)

_SF = _N * 16
_SFB = 102400
_SFNB = (_SF + _SFB - 1) // _SFB


def _make_step(c2, cm):
    return pl.pallas_call(
        functools.partial(_tc_step, c2=c2, cm=cm),
        grid=(_SFNB,),
        in_specs=[
            pl.BlockSpec((2, _SFB), lambda i: (0, i)),
            pl.BlockSpec((_SFB,), lambda i: (i,)),
            pl.BlockSpec((_SFB,), lambda i: (i,)),
            pl.BlockSpec((_SFB,), lambda i: (i,)),
        ],
        out_specs=[
            pl.BlockSpec((_SFB,), lambda i: (i,)),
            pl.BlockSpec((_SFB,), lambda i: (i,)),
        ],
        out_shape=[
            jax.ShapeDtypeStruct((_SF,), jnp.float32),
            jax.ShapeDtypeStruct((_SF,), jnp.float32),
        ],
    )


_step1_call = _make_step(1.0, 0.0)
_stepk_call = _make_step(2.0, 1.0)

_y_call = pl.pallas_call(
    _tc_y,
    grid=(_NB,),
    in_specs=[
        pl.BlockSpec((_K, 16, _P), lambda i: (0, 0, 0)),
    ] + [pl.BlockSpec((_BN, 16), lambda i: (i, 0)) for _ in range(_K)],
    out_specs=pl.BlockSpec((_BN, _P), lambda i: (i, 0)),
    out_shape=jax.ShapeDtypeStruct((_N, _P), jnp.float32),
)

_fc_call = pl.pallas_call(
    _tc_fc,
    grid=(_FNB,),
    in_specs=[
        pl.BlockSpec((_C,), lambda i: (0,)),
        pl.BlockSpec((_FCB,), lambda i: (i,)),
        pl.BlockSpec((_C, _FCB), lambda i: (0, i)),
    ],
    out_specs=pl.BlockSpec((_C,), lambda i: (0,)),
    out_shape=jax.ShapeDtypeStruct((_C,), jnp.float32),
)


def _build_mats(Wr, Wi):
    h = np.arange(_H, dtype=np.float64)[:, None]
    f = np.arange(_FQ, dtype=np.float64)[None, :]
    ang = 2.0 * np.pi * h * f / _H
    dmat = jnp.asarray(
        np.concatenate([np.cos(ang), -np.sin(ang)], axis=1), jnp.float32)

    ang2 = ang.T
    w2 = np.where(np.arange(_FQ)[:, None] == 0, 1.0, 2.0)
    cr = jnp.asarray((w2 * np.cos(ang2)) / _H, jnp.float32)
    ci_np = -(w2 * np.sin(ang2)) / _H
    ci_np[0, :] = 0.0
    ci = jnp.asarray(ci_np, jnp.float32)

    wr = Wr[:, :, 0, :]
    wi = Wi[:, :, 0, :]
    b_re = (jnp.einsum('fh,kfo->kfho', cr, wr)
            + jnp.einsum('fh,kfo->kfho', ci, wi))
    b_im = (-jnp.einsum('fh,kfo->kfho', cr, wi)
            + jnp.einsum('fh,kfo->kfho', ci, wr))
    bmat = jnp.concatenate([b_re, b_im], axis=1)
    return dmat, bmat.reshape(_K, 16, _P)


def kernel(x, edge_index, Wr, Wi, fc_w, fc_b):
    x2 = x[:, :, 0]
    src = edge_index[0]
    dst = edge_index[1]
    dmat, bmat = _build_mats(Wr, Wi)

    degp = _deg_call(src, dst).reshape(_NC, 2, _N, 1)
    rs16o, rs16i, t0, u = _prep_call(degp, x2, dmat)
    rsof, rsif = rs16o.reshape(-1), rs16i.reshape(-1)

    tfs = [t0.reshape(-1)]
    for k in range(1, _K):
        p = _spmv_call(u, src, dst)
        call = _step1_call if k == 1 else _stepk_call
        tf, uf = call(p.reshape(_NC, -1), rsif, rsof,
                      tfs[-2] if k >= 2 else tfs[0])
        tfs.append(tf)
        u = uf.reshape(_N, 16)

    ts = [t0] + [tf.reshape(_N, 16) for tf in tfs[1:]]
    y = _y_call(bmat, *ts)
    return _fc_call(fc_b, y.reshape(-1), fc_w)
